# baseline (speedup 1.0000x reference)
import functools

import jax
import jax.numpy as jnp
from jax import lax
from jax.experimental import pallas as pl
from jax.experimental.pallas import tpu as pltpu
from jax.experimental.pallas import tpu_sc as plsc

NUM_CORES = 2
NUM_SUBCORES = 16
NW = NUM_CORES * NUM_SUBCORES
IDX_CHUNK = 128


def _sc_gather_pair(user_id, item_id, user_table, item_table):
    B = user_id.shape[0]
    D = user_table.shape[1]
    bpw = B // NW
    nchunk = bpw // IDX_CHUNK
    mesh = plsc.VectorSubcoreMesh(core_axis_name="c", subcore_axis_name="s")

    @functools.partial(
        pl.kernel,
        mesh=mesh,
        compiler_params=pltpu.CompilerParams(use_tc_tiling_on_sc=False),
        out_type=[
            jax.ShapeDtypeStruct((B, D), jnp.float32),
            jax.ShapeDtypeStruct((B, D), jnp.float32),
        ],
        scratch_types=[
            pltpu.VMEM((nchunk, IDX_CHUNK), jnp.int32),
            pltpu.VMEM((nchunk, IDX_CHUNK), jnp.int32),
            pltpu.VMEM((bpw, D), jnp.float32),
            pltpu.VMEM((bpw, D), jnp.float32),
            pltpu.SemaphoreType.DMA,
        ],
    )
    def gather_kernel(uid_hbm, iid_hbm, ut_hbm, it_hbm, uout_hbm, iout_hbm,
                      uidx_v, iidx_v, urows_v, irows_v, sem):
        wid = lax.axis_index("s") * NUM_CORES + lax.axis_index("c")
        base = wid * bpw
        for j in range(nchunk):
            pltpu.sync_copy(uid_hbm.at[pl.ds(base + j * IDX_CHUNK, IDX_CHUNK)],
                            uidx_v.at[j])
            pltpu.sync_copy(iid_hbm.at[pl.ds(base + j * IDX_CHUNK, IDX_CHUNK)],
                            iidx_v.at[j])
        copies = []
        for j in range(nchunk):
            copies.append(pltpu.async_copy(
                ut_hbm.at[uidx_v.at[j]],
                urows_v.at[pl.ds(j * IDX_CHUNK, IDX_CHUNK)], sem))
            copies.append(pltpu.async_copy(
                it_hbm.at[iidx_v.at[j]],
                irows_v.at[pl.ds(j * IDX_CHUNK, IDX_CHUNK)], sem))
        for c in copies:
            c.wait()
        pltpu.sync_copy(urows_v, uout_hbm.at[pl.ds(base, bpw)])
        pltpu.sync_copy(irows_v, iout_hbm.at[pl.ds(base, bpw)])

    return gather_kernel(user_id, item_id, user_table, item_table)


def _mlp_body(u_ref, i_ref, p_ref, w1_ref, b1_ref, w2_ref, b2_ref, o_ref):
    w1 = w1_ref[...]
    D = u_ref.shape[1]
    h = (jnp.dot(u_ref[...], w1[0:D, :], precision=lax.Precision.HIGHEST,
                 preferred_element_type=jnp.float32)
         + jnp.dot(i_ref[...], w1[D:2 * D, :], precision=lax.Precision.HIGHEST,
                   preferred_element_type=jnp.float32)
         + p_ref[...] * w1[2 * D:2 * D + 1, :]
         + b1_ref[...])
    h = jnp.maximum(h, 0.0)
    z = jnp.dot(h, w2_ref[...], precision=lax.Precision.HIGHEST,
                preferred_element_type=jnp.float32) + b2_ref[...]
    o_ref[...] = jax.nn.sigmoid(z)


def kernel(user_id, item_id, price, user_table, item_table, W1, b1, W2, b2):
    B = user_id.shape[0]
    D = user_table.shape[1]
    H = W1.shape[1]
    u_emb, i_emb = _sc_gather_pair(user_id, item_id, user_table, item_table)

    blk = 2048
    grid = (B // blk,)
    out = pl.pallas_call(
        _mlp_body,
        grid=grid,
        in_specs=[
            pl.BlockSpec((blk, D), lambda i: (i, 0)),
            pl.BlockSpec((blk, D), lambda i: (i, 0)),
            pl.BlockSpec((blk, 1), lambda i: (i, 0)),
            pl.BlockSpec((2 * D + 1, H), lambda i: (0, 0)),
            pl.BlockSpec((1, H), lambda i: (0, 0)),
            pl.BlockSpec((H, 1), lambda i: (0, 0)),
            pl.BlockSpec((1, 1), lambda i: (0, 0)),
        ],
        out_specs=pl.BlockSpec((blk, 1), lambda i: (i, 0)),
        out_shape=jax.ShapeDtypeStruct((B, 1), jnp.float32),
    )(u_emb, i_emb, price.reshape(B, 1), W1, b1.reshape(1, H),
      W2, b2.reshape(1, 1))
    return out.reshape(B)
